# baseline (speedup 1.0000x reference)
import functools

import jax
import jax.numpy as jnp
from jax import lax
from jax.experimental import pallas as pl
from jax.experimental.pallas import tpu as pltpu
from jax.experimental.pallas import tpu_sc as plsc

HID = 256
NB = 10000
NCN = 1000
DE = 16
L = 2
CONVS = [("bb", "base", "base"), ("bc", "base", "centroid"),
         ("cc", "centroid", "centroid"), ("cb", "centroid", "base")]
EDGE_N = {"bb": 160000, "bc": 40000, "cc": 32000, "cb": 40000}

SC_CORES = 2
SC_SUBCORES = 16
CHUNK = 128
PAD_UNIT = SC_CORES * SC_SUBCORES * CHUNK

NPAD = {"base": 10240, "centroid": 1024}
NNODE = {"base": NB, "centroid": NCN}


def _ceil_to(x, m):
    return (x + m - 1) // m * m


def _lin_body(x_ref, w_ref, b_ref, o_ref):
    y = (jnp.dot(x_ref[...], w_ref[...],
                 preferred_element_type=jnp.float32) + b_ref[...])
    o_ref[...] = y.astype(o_ref.dtype)


def _lin(x, w, b, block_rows, out_dtype=jnp.float32):
    n, k = x.shape
    m = w.shape[1]
    return pl.pallas_call(
        _lin_body,
        grid=(n // block_rows,),
        in_specs=[pl.BlockSpec((block_rows, k), lambda i: (i, 0)),
                  pl.BlockSpec((k, m), lambda i: (0, 0)),
                  pl.BlockSpec((1, m), lambda i: (0, 0))],
        out_specs=pl.BlockSpec((block_rows, m), lambda i: (i, 0)),
        out_shape=jax.ShapeDtypeStruct((n, m), out_dtype),
    )(x, w, b.reshape(1, m))


def _msg_body(g_ref, ea_ref, ew_ref, w_ref, b_ref, o_ref):
    pre = (g_ref[...].astype(jnp.float32)
           + jnp.dot(ea_ref[...], w_ref[...],
                     preferred_element_type=jnp.float32) + b_ref[...])
    o_ref[...] = jax.nn.gelu(pre) * ew_ref[...]


def _msg(g, ea, ew, w, b, block_rows):
    ep = g.shape[0]
    return pl.pallas_call(
        _msg_body,
        grid=(ep // block_rows,),
        in_specs=[pl.BlockSpec((block_rows, HID), lambda i: (i, 0)),
                  pl.BlockSpec((block_rows, DE), lambda i: (i, 0)),
                  pl.BlockSpec((block_rows, 1), lambda i: (i, 0)),
                  pl.BlockSpec((DE, HID), lambda i: (0, 0)),
                  pl.BlockSpec((1, HID), lambda i: (0, 0))],
        out_specs=pl.BlockSpec((block_rows, HID), lambda i: (i, 0)),
        out_shape=jax.ShapeDtypeStruct((ep, HID), jnp.float32),
    )(g, ea, ew, w, b.reshape(1, HID))


def _post_body(a_ref, h_ref, m0_ref, b0_ref, g_ref, be_ref, m1_ref, b1_ref,
               o_ref):
    h = a_ref[...] + h_ref[...]
    h = jnp.dot(h, m0_ref[...], preferred_element_type=jnp.float32) + b0_ref[...]
    mu = jnp.mean(h, -1, keepdims=True)
    v = jnp.mean((h - mu) ** 2, -1, keepdims=True)
    h = (h - mu) / jnp.sqrt(v + 1e-5) * g_ref[...] + be_ref[...]
    h = jax.nn.gelu(h)
    o_ref[...] = (jnp.dot(h, m1_ref[...], preferred_element_type=jnp.float32)
                  + b1_ref[...])


def _post(aggr, hd, m0, b0, lg, lb, m1, b1, block_rows):
    n = aggr.shape[0]
    full = lambda i: (0, 0)
    rows = lambda i: (i, 0)
    return pl.pallas_call(
        _post_body,
        grid=(n // block_rows,),
        in_specs=[pl.BlockSpec((block_rows, HID), rows),
                  pl.BlockSpec((block_rows, HID), rows),
                  pl.BlockSpec((HID, HID), full),
                  pl.BlockSpec((1, HID), full),
                  pl.BlockSpec((1, HID), full),
                  pl.BlockSpec((1, HID), full),
                  pl.BlockSpec((HID, HID), full),
                  pl.BlockSpec((1, HID), full)],
        out_specs=pl.BlockSpec((block_rows, HID), rows),
        out_shape=jax.ShapeDtypeStruct((n, HID), jnp.float32),
    )(aggr, hd, m0, b0.reshape(1, HID), lg.reshape(1, HID),
      lb.reshape(1, HID), m1, b1.reshape(1, HID))


def _combine_body(x_ref, a_ref, b_ref, o_ref):
    o_ref[...] = x_ref[...] + jax.nn.gelu(a_ref[...] + b_ref[...])


def _combine(x, a, b, block_rows):
    n = x.shape[0]
    rows = lambda i: (i, 0)
    return pl.pallas_call(
        _combine_body,
        grid=(n // block_rows,),
        in_specs=[pl.BlockSpec((block_rows, HID), rows)] * 3,
        out_specs=pl.BlockSpec((block_rows, HID), rows),
        out_shape=jax.ShapeDtypeStruct((n, HID), jnp.float32),
    )(x, a, b)


def _make_gather(ep):
    per_worker = ep // (SC_CORES * SC_SUBCORES)
    niter = per_worker // CHUNK
    mesh = plsc.VectorSubcoreMesh(core_axis_name="c", subcore_axis_name="s")

    @functools.partial(
        pl.kernel,
        out_type=jax.ShapeDtypeStruct((ep, HID // 2), jnp.int32),
        mesh=mesh,
        scratch_types=[pltpu.VMEM((per_worker,), jnp.int32),
                       pltpu.VMEM((CHUNK, HID // 2), jnp.int32),
                       pltpu.VMEM((CHUNK, HID // 2), jnp.int32),
                       pltpu.SemaphoreType.DMA,
                       pltpu.SemaphoreType.DMA,
                       pltpu.SemaphoreType.DMA,
                       pltpu.SemaphoreType.DMA],
    )
    def gather_k(table_hbm, idx_hbm, out_hbm, idx_v, rows_a, rows_b,
                 sga, sgb, swa, swb):
        wid = lax.axis_index("s") * SC_CORES + lax.axis_index("c")
        base = wid * per_worker
        pltpu.sync_copy(idx_hbm.at[pl.ds(base, per_worker)], idx_v)

        @pl.loop(0, niter // 2)
        def _(k):
            c0 = 2 * k * CHUNK
            c1 = c0 + CHUNK

            @pl.when(k > 0)
            def _():
                pltpu.make_async_copy(rows_a, out_hbm.at[pl.ds(base, CHUNK)],
                                      swa).wait()
                pltpu.make_async_copy(rows_b, out_hbm.at[pl.ds(base, CHUNK)],
                                      swb).wait()

            ha = pltpu.async_copy(table_hbm.at[idx_v.at[pl.ds(c0, CHUNK)]],
                                  rows_a, sga)
            hb = pltpu.async_copy(table_hbm.at[idx_v.at[pl.ds(c1, CHUNK)]],
                                  rows_b, sgb)
            ha.wait()
            pltpu.async_copy(rows_a, out_hbm.at[pl.ds(base + c0, CHUNK)], swa)
            hb.wait()
            pltpu.async_copy(rows_b, out_hbm.at[pl.ds(base + c1, CHUNK)], swb)

        pltpu.make_async_copy(rows_a, out_hbm.at[pl.ds(base, CHUNK)],
                              swa).wait()
        pltpu.make_async_copy(rows_b, out_hbm.at[pl.ds(base, CHUNK)],
                              swb).wait()

    return gather_k


def _make_scatter(nd_pad, ep):
    half = HID // 2
    e_sub = ep // SC_SUBCORES
    niter = e_sub // CHUNK
    rows_sub = nd_pad // SC_SUBCORES
    mesh = plsc.VectorSubcoreMesh(core_axis_name="c", subcore_axis_name="s")

    @functools.partial(
        pl.kernel,
        out_type=jax.ShapeDtypeStruct((nd_pad, HID), jnp.float32),
        mesh=mesh,
        scratch_types=[pltpu.VMEM((CHUNK,), jnp.int32),
                       pltpu.VMEM((CHUNK,), jnp.int32),
                       pltpu.VMEM((CHUNK, half), jnp.float32),
                       pltpu.VMEM((CHUNK, half), jnp.float32),
                       pltpu.VMEM_SHARED((nd_pad, half), jnp.float32),
                       pltpu.SemaphoreType.DMA,
                       pltpu.SemaphoreType.DMA,
                       pltpu.SemaphoreType.DMA,
                       pltpu.SemaphoreType.DMA,
                       pltpu.SemaphoreType.DMA,
                       pltpu.SemaphoreType.DMA],
    )
    def scatter_k(m_hbm, di_hbm, zeros_hbm, out_hbm, idx_a, idx_b, rows_a,
                  rows_b, acc_sh, sia, sib, sla, slb, ssa, ssb):
        cid = lax.axis_index("c")
        sid = lax.axis_index("s")
        r0 = sid * rows_sub
        col0 = cid * half
        pltpu.sync_copy(zeros_hbm.at[pl.ds(r0, rows_sub)],
                        acc_sh.at[pl.ds(r0, rows_sub)])
        plsc.subcore_barrier()

        @pl.loop(0, niter // 2)
        def _(k):
            off = sid * e_sub + 2 * k * CHUNK
            hia = pltpu.async_copy(di_hbm.at[pl.ds(off, CHUNK)], idx_a, sia)
            hla = pltpu.async_copy(
                m_hbm.at[pl.ds(off, CHUNK), pl.ds(col0, half)], rows_a, sla)
            hib = pltpu.async_copy(di_hbm.at[pl.ds(off + CHUNK, CHUNK)],
                                   idx_b, sib)
            hlb = pltpu.async_copy(
                m_hbm.at[pl.ds(off + CHUNK, CHUNK), pl.ds(col0, half)],
                rows_b, slb)
            hia.wait()
            hla.wait()
            hsa = pltpu.async_copy(rows_a, acc_sh.at[idx_a], ssa, add=True)
            hib.wait()
            hlb.wait()
            hsb = pltpu.async_copy(rows_b, acc_sh.at[idx_b], ssb, add=True)
            hsa.wait()
            hsb.wait()

        plsc.subcore_barrier()
        pltpu.sync_copy(acc_sh.at[pl.ds(r0, rows_sub)],
                        out_hbm.at[pl.ds(r0, rows_sub), pl.ds(col0, half)])

    return scatter_k


_EPAD = {k: _ceil_to(v, PAD_UNIT) for k, v in EDGE_N.items()}
_GATHER = {k: _make_gather(ep) for k, ep in _EPAD.items()}
_SCATTER = {name: _make_scatter(NPAD[dt], _EPAD[name])
            for name, _, dt in CONVS}


def kernel(x_base, x_centroid, edge_attr_bb, edge_attr_bc, edge_attr_cc,
           edge_attr_cb, edge_weight_bb, edge_weight_bc, edge_weight_cc,
           edge_weight_cb, src_bb, dst_bb, src_bc, dst_bc, src_cc, dst_cc,
           src_cb, dst_cb, W_src, b_src, W_dst, b_dst, W_edge, b_edge, eps,
           W_mlp, b_mlp, ln_g, ln_b):
    ea = {"bb": edge_attr_bb, "bc": edge_attr_bc, "cc": edge_attr_cc,
          "cb": edge_attr_cb}
    ew = {"bb": edge_weight_bb, "bc": edge_weight_bc, "cc": edge_weight_cc,
          "cb": edge_weight_cb}
    si = {"bb": src_bb, "bc": src_bc, "cc": src_cc, "cb": src_cb}
    di = {"bb": dst_bb, "bc": dst_bc, "cc": dst_cc, "cb": dst_cb}

    for name in EDGE_N:
        e = EDGE_N[name]
        ep = _EPAD[name]
        pad = ep - e
        ea[name] = jnp.pad(ea[name], ((0, pad), (0, 0)))
        ew[name] = jnp.pad(ew[name], (0, pad)).reshape(ep, 1)
        si[name] = jnp.pad(si[name], (0, pad))
        di[name] = jnp.pad(di[name], (0, pad))

    zeros = {"base": jnp.zeros((NPAD["base"], HID // 2), jnp.float32),
             "centroid": jnp.zeros((NPAD["centroid"], HID // 2), jnp.float32)}

    xmap = {"base": x_base, "centroid": x_centroid}
    for l in range(L):
        outs = {"base": [], "centroid": []}
        for c, (name, st, dt) in enumerate(CONVS):
            xs = xmap[st]
            xd = xmap[dt]
            nd = NNODE[dt]
            blk_s = 1000 if xs.shape[0] == NB else NCN
            blk_d = 1000 if nd == NB else NCN
            hsrc = _lin(xs, W_src[l, c], b_src[l, c], blk_s, jnp.bfloat16)
            hsrc = jax.lax.bitcast_convert_type(
                hsrc.reshape(-1, HID // 2, 2), jnp.int32)
            hd = _lin(xd, W_dst[l, c] * (1.0 + eps[l, c]), b_dst[l, c], blk_d)
            g = jax.lax.bitcast_convert_type(
                _GATHER[name](hsrc, si[name]), jnp.bfloat16).reshape(-1, HID)
            m = _msg(g, ea[name], ew[name], W_edge[l, c], b_edge[l, c], 2048)
            aggr = _SCATTER[name](m, di[name], zeros[dt])[:nd]
            out = _post(aggr, hd, W_mlp[l, c, 0], b_mlp[l, c, 0], ln_g[l, c],
                        ln_b[l, c], W_mlp[l, c, 1], b_mlp[l, c, 1], blk_d)
            outs[dt].append(out)
        xmap = {k: _combine(xmap[k], outs[k][0], outs[k][1],
                            1000 if NNODE[k] == NB else NCN)
                for k in xmap}
    return jnp.concatenate([xmap["base"], xmap["centroid"]], axis=0)


# speedup vs baseline: 2.2245x; 2.2245x over previous
import functools

import jax
import jax.numpy as jnp
from jax import lax
from jax.experimental import pallas as pl
from jax.experimental.pallas import tpu as pltpu
from jax.experimental.pallas import tpu_sc as plsc

HID = 256
NB = 10000
NCN = 1000
DE = 16
L = 2
CONVS = [("bb", "base", "base"), ("bc", "base", "centroid"),
         ("cc", "centroid", "centroid"), ("cb", "centroid", "base")]
EDGE_N = {"bb": 160000, "bc": 40000, "cc": 32000, "cb": 40000}

SC_CORES = 2
SC_SUBCORES = 16
CHUNK = 128
PAD_UNIT = SC_CORES * SC_SUBCORES * CHUNK

NPAD = {"base": 10240, "centroid": 1024}
NNODE = {"base": NB, "centroid": NCN}


def _ceil_to(x, m):
    return (x + m - 1) // m * m


def _lin_body(x_ref, w_ref, b_ref, o_ref):
    y = (jnp.dot(x_ref[...], w_ref[...],
                 preferred_element_type=jnp.float32) + b_ref[...])
    o_ref[...] = y.astype(o_ref.dtype)


def _lin(x, w, b, block_rows, out_dtype=jnp.float32):
    n, k = x.shape
    m = w.shape[1]
    return pl.pallas_call(
        _lin_body,
        grid=(n // block_rows,),
        in_specs=[pl.BlockSpec((block_rows, k), lambda i: (i, 0)),
                  pl.BlockSpec((k, m), lambda i: (0, 0)),
                  pl.BlockSpec((1, m), lambda i: (0, 0))],
        out_specs=pl.BlockSpec((block_rows, m), lambda i: (i, 0)),
        out_shape=jax.ShapeDtypeStruct((n, m), out_dtype),
    )(x, w, b.reshape(1, m))


def _pack_bf16_pair(lo, hi):
    def rne(x):
        u = jax.lax.bitcast_convert_type(x, jnp.uint32)
        return (u + 0x7FFF + ((u >> 16) & 1)) >> 16
    return jax.lax.bitcast_convert_type(rne(lo) | (rne(hi) << 16), jnp.int32)


def _unpack_bf16_pair(p):
    u = jax.lax.bitcast_convert_type(p, jnp.uint32)
    lo = jax.lax.bitcast_convert_type(u << 16, jnp.float32)
    hi = jax.lax.bitcast_convert_type(u & jnp.uint32(0xFFFF0000), jnp.float32)
    return lo, hi


def _lin_packed_body(x_ref, w_ref, b_ref, o_ref):
    y = (jnp.dot(x_ref[...], w_ref[...],
                 preferred_element_type=jnp.float32) + b_ref[...])
    half = y.shape[1] // 2
    o_ref[...] = _pack_bf16_pair(y[:, :half], y[:, half:])


def _lin_packed(x, w, b, block_rows):
    n, k = x.shape
    m = w.shape[1]
    return pl.pallas_call(
        _lin_packed_body,
        grid=(n // block_rows,),
        in_specs=[pl.BlockSpec((block_rows, k), lambda i: (i, 0)),
                  pl.BlockSpec((k, m), lambda i: (0, 0)),
                  pl.BlockSpec((1, m), lambda i: (0, 0))],
        out_specs=pl.BlockSpec((block_rows, m // 2), lambda i: (i, 0)),
        out_shape=jax.ShapeDtypeStruct((n, m // 2), jnp.int32),
    )(x, w, b.reshape(1, m))


def _msg_body(g_ref, ea_ref, ew_ref, w_ref, b_ref, o_ref):
    g_lo, g_hi = _unpack_bf16_pair(g_ref[...])
    pre = (jnp.concatenate([g_lo, g_hi], axis=1)
           + jnp.dot(ea_ref[...], w_ref[...],
                     preferred_element_type=jnp.float32) + b_ref[...])
    o_ref[...] = jax.nn.gelu(pre) * ew_ref[...]


def _msg(g, ea, ew, w, b, block_rows):
    ep = g.shape[0]
    return pl.pallas_call(
        _msg_body,
        grid=(ep // block_rows,),
        in_specs=[pl.BlockSpec((block_rows, HID // 2), lambda i: (i, 0)),
                  pl.BlockSpec((block_rows, DE), lambda i: (i, 0)),
                  pl.BlockSpec((block_rows, 1), lambda i: (i, 0)),
                  pl.BlockSpec((DE, HID), lambda i: (0, 0)),
                  pl.BlockSpec((1, HID), lambda i: (0, 0))],
        out_specs=pl.BlockSpec((block_rows, HID), lambda i: (i, 0)),
        out_shape=jax.ShapeDtypeStruct((ep, HID), jnp.float32),
    )(g, ea, ew, w, b.reshape(1, HID))


def _post_body(a_ref, h_ref, m0_ref, b0_ref, g_ref, be_ref, m1_ref, b1_ref,
               o_ref):
    h = a_ref[...] + h_ref[...]
    h = jnp.dot(h, m0_ref[...], preferred_element_type=jnp.float32) + b0_ref[...]
    mu = jnp.mean(h, -1, keepdims=True)
    v = jnp.mean((h - mu) ** 2, -1, keepdims=True)
    h = (h - mu) / jnp.sqrt(v + 1e-5) * g_ref[...] + be_ref[...]
    h = jax.nn.gelu(h)
    o_ref[...] = (jnp.dot(h, m1_ref[...], preferred_element_type=jnp.float32)
                  + b1_ref[...])


def _post(aggr, hd, m0, b0, lg, lb, m1, b1, block_rows):
    n = aggr.shape[0]
    full = lambda i: (0, 0)
    rows = lambda i: (i, 0)
    return pl.pallas_call(
        _post_body,
        grid=(n // block_rows,),
        in_specs=[pl.BlockSpec((block_rows, HID), rows),
                  pl.BlockSpec((block_rows, HID), rows),
                  pl.BlockSpec((HID, HID), full),
                  pl.BlockSpec((1, HID), full),
                  pl.BlockSpec((1, HID), full),
                  pl.BlockSpec((1, HID), full),
                  pl.BlockSpec((HID, HID), full),
                  pl.BlockSpec((1, HID), full)],
        out_specs=pl.BlockSpec((block_rows, HID), rows),
        out_shape=jax.ShapeDtypeStruct((n, HID), jnp.float32),
    )(aggr, hd, m0, b0.reshape(1, HID), lg.reshape(1, HID),
      lb.reshape(1, HID), m1, b1.reshape(1, HID))


def _combine_body(x_ref, a_ref, b_ref, o_ref):
    o_ref[...] = x_ref[...] + jax.nn.gelu(a_ref[...] + b_ref[...])


def _combine(x, a, b, block_rows):
    n = x.shape[0]
    rows = lambda i: (i, 0)
    return pl.pallas_call(
        _combine_body,
        grid=(n // block_rows,),
        in_specs=[pl.BlockSpec((block_rows, HID), rows)] * 3,
        out_specs=pl.BlockSpec((block_rows, HID), rows),
        out_shape=jax.ShapeDtypeStruct((n, HID), jnp.float32),
    )(x, a, b)


def _make_gather(ep):
    per_worker = ep // (SC_CORES * SC_SUBCORES)
    niter = per_worker // CHUNK
    mesh = plsc.VectorSubcoreMesh(core_axis_name="c", subcore_axis_name="s")

    @functools.partial(
        pl.kernel,
        out_type=jax.ShapeDtypeStruct((ep, HID // 2), jnp.int32),
        mesh=mesh,
        scratch_types=[pltpu.VMEM((per_worker,), jnp.int32),
                       pltpu.VMEM((CHUNK, HID // 2), jnp.int32),
                       pltpu.VMEM((CHUNK, HID // 2), jnp.int32),
                       pltpu.SemaphoreType.DMA,
                       pltpu.SemaphoreType.DMA,
                       pltpu.SemaphoreType.DMA,
                       pltpu.SemaphoreType.DMA],
    )
    def gather_k(table_hbm, idx_hbm, out_hbm, idx_v, rows_a, rows_b,
                 sga, sgb, swa, swb):
        wid = lax.axis_index("s") * SC_CORES + lax.axis_index("c")
        base = wid * per_worker
        pltpu.sync_copy(idx_hbm.at[pl.ds(base, per_worker)], idx_v)

        @pl.loop(0, niter // 2)
        def _(k):
            c0 = 2 * k * CHUNK
            c1 = c0 + CHUNK

            @pl.when(k > 0)
            def _():
                pltpu.make_async_copy(rows_a, out_hbm.at[pl.ds(base, CHUNK)],
                                      swa).wait()
                pltpu.make_async_copy(rows_b, out_hbm.at[pl.ds(base, CHUNK)],
                                      swb).wait()

            ha = pltpu.async_copy(table_hbm.at[idx_v.at[pl.ds(c0, CHUNK)]],
                                  rows_a, sga)
            hb = pltpu.async_copy(table_hbm.at[idx_v.at[pl.ds(c1, CHUNK)]],
                                  rows_b, sgb)
            ha.wait()
            pltpu.async_copy(rows_a, out_hbm.at[pl.ds(base + c0, CHUNK)], swa)
            hb.wait()
            pltpu.async_copy(rows_b, out_hbm.at[pl.ds(base + c1, CHUNK)], swb)

        pltpu.make_async_copy(rows_a, out_hbm.at[pl.ds(base, CHUNK)],
                              swa).wait()
        pltpu.make_async_copy(rows_b, out_hbm.at[pl.ds(base, CHUNK)],
                              swb).wait()

    return gather_k


def _make_scatter(nd_pad, ep):
    half = HID // 2
    e_sub = ep // SC_SUBCORES
    niter = e_sub // CHUNK
    rows_sub = nd_pad // SC_SUBCORES
    mesh = plsc.VectorSubcoreMesh(core_axis_name="c", subcore_axis_name="s")

    @functools.partial(
        pl.kernel,
        out_type=jax.ShapeDtypeStruct((nd_pad, HID), jnp.float32),
        mesh=mesh,
        scratch_types=[pltpu.VMEM((CHUNK,), jnp.int32),
                       pltpu.VMEM((CHUNK,), jnp.int32),
                       pltpu.VMEM((CHUNK, half), jnp.float32),
                       pltpu.VMEM((CHUNK, half), jnp.float32),
                       pltpu.VMEM_SHARED((nd_pad, half), jnp.float32),
                       pltpu.SemaphoreType.DMA,
                       pltpu.SemaphoreType.DMA,
                       pltpu.SemaphoreType.DMA,
                       pltpu.SemaphoreType.DMA,
                       pltpu.SemaphoreType.DMA,
                       pltpu.SemaphoreType.DMA],
    )
    def scatter_k(m_hbm, di_hbm, zeros_hbm, out_hbm, idx_a, idx_b, rows_a,
                  rows_b, acc_sh, sia, sib, sla, slb, ssa, ssb):
        cid = lax.axis_index("c")
        sid = lax.axis_index("s")
        r0 = sid * rows_sub
        col0 = cid * half
        pltpu.sync_copy(zeros_hbm.at[pl.ds(r0, rows_sub)],
                        acc_sh.at[pl.ds(r0, rows_sub)])
        plsc.subcore_barrier()

        @pl.loop(0, niter // 2)
        def _(k):
            off = sid * e_sub + 2 * k * CHUNK
            hia = pltpu.async_copy(di_hbm.at[pl.ds(off, CHUNK)], idx_a, sia)
            hla = pltpu.async_copy(
                m_hbm.at[pl.ds(off, CHUNK), pl.ds(col0, half)], rows_a, sla)
            hib = pltpu.async_copy(di_hbm.at[pl.ds(off + CHUNK, CHUNK)],
                                   idx_b, sib)
            hlb = pltpu.async_copy(
                m_hbm.at[pl.ds(off + CHUNK, CHUNK), pl.ds(col0, half)],
                rows_b, slb)
            hia.wait()
            hla.wait()
            hsa = pltpu.async_copy(rows_a, acc_sh.at[idx_a], ssa, add=True)
            hib.wait()
            hlb.wait()
            hsb = pltpu.async_copy(rows_b, acc_sh.at[idx_b], ssb, add=True)
            hsa.wait()
            hsb.wait()

        plsc.subcore_barrier()
        pltpu.sync_copy(acc_sh.at[pl.ds(r0, rows_sub)],
                        out_hbm.at[pl.ds(r0, rows_sub), pl.ds(col0, half)])

    return scatter_k


_EPAD = {k: _ceil_to(v, PAD_UNIT) for k, v in EDGE_N.items()}
_GATHER = {k: _make_gather(ep) for k, ep in _EPAD.items()}
_SCATTER = {name: _make_scatter(NPAD[dt], _EPAD[name])
            for name, _, dt in CONVS}


def kernel(x_base, x_centroid, edge_attr_bb, edge_attr_bc, edge_attr_cc,
           edge_attr_cb, edge_weight_bb, edge_weight_bc, edge_weight_cc,
           edge_weight_cb, src_bb, dst_bb, src_bc, dst_bc, src_cc, dst_cc,
           src_cb, dst_cb, W_src, b_src, W_dst, b_dst, W_edge, b_edge, eps,
           W_mlp, b_mlp, ln_g, ln_b):
    ea = {"bb": edge_attr_bb, "bc": edge_attr_bc, "cc": edge_attr_cc,
          "cb": edge_attr_cb}
    ew = {"bb": edge_weight_bb, "bc": edge_weight_bc, "cc": edge_weight_cc,
          "cb": edge_weight_cb}
    si = {"bb": src_bb, "bc": src_bc, "cc": src_cc, "cb": src_cb}
    di = {"bb": dst_bb, "bc": dst_bc, "cc": dst_cc, "cb": dst_cb}

    for name in EDGE_N:
        e = EDGE_N[name]
        ep = _EPAD[name]
        pad = ep - e
        ea[name] = jnp.pad(ea[name], ((0, pad), (0, 0)))
        ew[name] = jnp.pad(ew[name], (0, pad)).reshape(ep, 1)
        si[name] = jnp.pad(si[name], (0, pad))
        di[name] = jnp.pad(di[name], (0, pad))

    zeros = {"base": jnp.zeros((NPAD["base"], HID // 2), jnp.float32),
             "centroid": jnp.zeros((NPAD["centroid"], HID // 2), jnp.float32)}

    xmap = {"base": x_base, "centroid": x_centroid}
    for l in range(L):
        outs = {"base": [], "centroid": []}
        for c, (name, st, dt) in enumerate(CONVS):
            xs = xmap[st]
            xd = xmap[dt]
            nd = NNODE[dt]
            blk_s = 1000 if xs.shape[0] == NB else NCN
            blk_d = 1000 if nd == NB else NCN
            hsrc = _lin_packed(xs, W_src[l, c], b_src[l, c], blk_s)
            hd = _lin(xd, W_dst[l, c] * (1.0 + eps[l, c]), b_dst[l, c], blk_d)
            g = _GATHER[name](hsrc, si[name])
            m = _msg(g, ea[name], ew[name], W_edge[l, c], b_edge[l, c], 2048)
            aggr = _SCATTER[name](m, di[name], zeros[dt])[:nd]
            out = _post(aggr, hd, W_mlp[l, c, 0], b_mlp[l, c, 0], ln_g[l, c],
                        ln_b[l, c], W_mlp[l, c, 1], b_mlp[l, c, 1], blk_d)
            outs[dt].append(out)
        xmap = {k: _combine(xmap[k], outs[k][0], outs[k][1],
                            1000 if NNODE[k] == NB else NCN)
                for k in xmap}
    return jnp.concatenate([xmap["base"], xmap["centroid"]], axis=0)


# speedup vs baseline: 2.2269x; 1.0011x over previous
import functools

import jax
import jax.numpy as jnp
from jax import lax
from jax.experimental import pallas as pl
from jax.experimental.pallas import tpu as pltpu
from jax.experimental.pallas import tpu_sc as plsc

HID = 256
NB = 10000
NCN = 1000
DE = 16
L = 2
CONVS = [("bb", "base", "base"), ("bc", "base", "centroid"),
         ("cc", "centroid", "centroid"), ("cb", "centroid", "base")]
EDGE_N = {"bb": 160000, "bc": 40000, "cc": 32000, "cb": 40000}

SC_CORES = 2
SC_SUBCORES = 16
CHUNK = 128
PAD_UNIT = SC_CORES * SC_SUBCORES * CHUNK

NPAD = {"base": 10240, "centroid": 1024}
NNODE = {"base": NB, "centroid": NCN}


def _ceil_to(x, m):
    return (x + m - 1) // m * m


def _lin_body(x_ref, w_ref, b_ref, o_ref):
    y = (jnp.dot(x_ref[...], w_ref[...],
                 preferred_element_type=jnp.float32) + b_ref[...])
    o_ref[...] = y.astype(o_ref.dtype)


def _lin(x, w, b, block_rows, out_dtype=jnp.float32):
    n, k = x.shape
    m = w.shape[1]
    return pl.pallas_call(
        _lin_body,
        grid=(n // block_rows,),
        in_specs=[pl.BlockSpec((block_rows, k), lambda i: (i, 0)),
                  pl.BlockSpec((k, m), lambda i: (0, 0)),
                  pl.BlockSpec((1, m), lambda i: (0, 0))],
        out_specs=pl.BlockSpec((block_rows, m), lambda i: (i, 0)),
        out_shape=jax.ShapeDtypeStruct((n, m), out_dtype),
    )(x, w, b.reshape(1, m))


def _pack_bf16_pair(lo, hi):
    def rne(x):
        u = jax.lax.bitcast_convert_type(x, jnp.uint32)
        return (u + 0x7FFF + ((u >> 16) & 1)) >> 16
    return jax.lax.bitcast_convert_type(rne(lo) | (rne(hi) << 16), jnp.int32)


def _unpack_bf16_pair(p):
    u = jax.lax.bitcast_convert_type(p, jnp.uint32)
    lo = jax.lax.bitcast_convert_type(u << 16, jnp.float32)
    hi = jax.lax.bitcast_convert_type(u & jnp.uint32(0xFFFF0000), jnp.float32)
    return lo, hi


def _lin_packed_body(x_ref, w_ref, b_ref, o_ref):
    y = (jnp.dot(x_ref[...], w_ref[...],
                 preferred_element_type=jnp.float32) + b_ref[...])
    half = y.shape[1] // 2
    o_ref[...] = _pack_bf16_pair(y[:, :half], y[:, half:])


def _lin_packed(x, w, b, block_rows):
    n, k = x.shape
    m = w.shape[1]
    return pl.pallas_call(
        _lin_packed_body,
        grid=(n // block_rows,),
        in_specs=[pl.BlockSpec((block_rows, k), lambda i: (i, 0)),
                  pl.BlockSpec((k, m), lambda i: (0, 0)),
                  pl.BlockSpec((1, m), lambda i: (0, 0))],
        out_specs=pl.BlockSpec((block_rows, m // 2), lambda i: (i, 0)),
        out_shape=jax.ShapeDtypeStruct((n, m // 2), jnp.int32),
    )(x, w, b.reshape(1, m))


def _msg_body(g_ref, ea_ref, ew_ref, w_ref, b_ref, o_ref):
    g_lo, g_hi = _unpack_bf16_pair(g_ref[...])
    pre = (jnp.concatenate([g_lo, g_hi], axis=1)
           + jnp.dot(ea_ref[...], w_ref[...],
                     preferred_element_type=jnp.float32) + b_ref[...])
    y = jax.nn.gelu(pre) * ew_ref[...]
    half = y.shape[1] // 2
    o_ref[0] = y[:, :half]
    o_ref[1] = y[:, half:]


def _msg(g, ea, ew, w, b, block_rows):
    ep = g.shape[0]
    return pl.pallas_call(
        _msg_body,
        grid=(ep // block_rows,),
        in_specs=[pl.BlockSpec((block_rows, HID // 2), lambda i: (i, 0)),
                  pl.BlockSpec((block_rows, DE), lambda i: (i, 0)),
                  pl.BlockSpec((block_rows, 1), lambda i: (i, 0)),
                  pl.BlockSpec((DE, HID), lambda i: (0, 0)),
                  pl.BlockSpec((1, HID), lambda i: (0, 0))],
        out_specs=pl.BlockSpec((2, block_rows, HID // 2), lambda i: (0, i, 0)),
        out_shape=jax.ShapeDtypeStruct((2, ep, HID // 2), jnp.float32),
    )(g, ea, ew, w, b.reshape(1, HID))


def _post_body(a0_ref, a1_ref, h_ref, m0_ref, b0_ref, g_ref, be_ref, m1_ref,
               b1_ref, o_ref):
    h = jnp.concatenate([a0_ref[...], a1_ref[...]], axis=1) + h_ref[...]
    h = jnp.dot(h, m0_ref[...], preferred_element_type=jnp.float32) + b0_ref[...]
    mu = jnp.mean(h, -1, keepdims=True)
    v = jnp.mean((h - mu) ** 2, -1, keepdims=True)
    h = (h - mu) / jnp.sqrt(v + 1e-5) * g_ref[...] + be_ref[...]
    h = jax.nn.gelu(h)
    o_ref[...] = (jnp.dot(h, m1_ref[...], preferred_element_type=jnp.float32)
                  + b1_ref[...])


def _post(a0, a1, hd, m0, b0, lg, lb, m1, b1, block_rows):
    n = a0.shape[0]
    full = lambda i: (0, 0)
    rows = lambda i: (i, 0)
    return pl.pallas_call(
        _post_body,
        grid=(n // block_rows,),
        in_specs=[pl.BlockSpec((block_rows, HID // 2), rows),
                  pl.BlockSpec((block_rows, HID // 2), rows),
                  pl.BlockSpec((block_rows, HID), rows),
                  pl.BlockSpec((HID, HID), full),
                  pl.BlockSpec((1, HID), full),
                  pl.BlockSpec((1, HID), full),
                  pl.BlockSpec((1, HID), full),
                  pl.BlockSpec((HID, HID), full),
                  pl.BlockSpec((1, HID), full)],
        out_specs=pl.BlockSpec((block_rows, HID), rows),
        out_shape=jax.ShapeDtypeStruct((n, HID), jnp.float32),
    )(a0, a1, hd, m0, b0.reshape(1, HID), lg.reshape(1, HID),
      lb.reshape(1, HID), m1, b1.reshape(1, HID))


def _combine_body(x_ref, a_ref, b_ref, o_ref):
    o_ref[...] = x_ref[...] + jax.nn.gelu(a_ref[...] + b_ref[...])


def _combine(x, a, b, block_rows):
    n = x.shape[0]
    rows = lambda i: (i, 0)
    return pl.pallas_call(
        _combine_body,
        grid=(n // block_rows,),
        in_specs=[pl.BlockSpec((block_rows, HID), rows)] * 3,
        out_specs=pl.BlockSpec((block_rows, HID), rows),
        out_shape=jax.ShapeDtypeStruct((n, HID), jnp.float32),
    )(x, a, b)


def _make_gather(ep):
    per_worker = ep // (SC_CORES * SC_SUBCORES)
    niter = per_worker // CHUNK
    mesh = plsc.VectorSubcoreMesh(core_axis_name="c", subcore_axis_name="s")

    @functools.partial(
        pl.kernel,
        out_type=jax.ShapeDtypeStruct((ep, HID // 2), jnp.int32),
        mesh=mesh,
        scratch_types=[pltpu.VMEM((per_worker,), jnp.int32),
                       pltpu.VMEM((CHUNK, HID // 2), jnp.int32),
                       pltpu.VMEM((CHUNK, HID // 2), jnp.int32),
                       pltpu.SemaphoreType.DMA,
                       pltpu.SemaphoreType.DMA,
                       pltpu.SemaphoreType.DMA,
                       pltpu.SemaphoreType.DMA],
    )
    def gather_k(table_hbm, idx_hbm, out_hbm, idx_v, rows_a, rows_b,
                 sga, sgb, swa, swb):
        wid = lax.axis_index("s") * SC_CORES + lax.axis_index("c")
        base = wid * per_worker
        pltpu.sync_copy(idx_hbm.at[pl.ds(base, per_worker)], idx_v)

        @pl.loop(0, niter // 2)
        def _(k):
            c0 = 2 * k * CHUNK
            c1 = c0 + CHUNK

            @pl.when(k > 0)
            def _():
                pltpu.make_async_copy(rows_a, out_hbm.at[pl.ds(base, CHUNK)],
                                      swa).wait()
                pltpu.make_async_copy(rows_b, out_hbm.at[pl.ds(base, CHUNK)],
                                      swb).wait()

            ha = pltpu.async_copy(table_hbm.at[idx_v.at[pl.ds(c0, CHUNK)]],
                                  rows_a, sga)
            hb = pltpu.async_copy(table_hbm.at[idx_v.at[pl.ds(c1, CHUNK)]],
                                  rows_b, sgb)
            ha.wait()
            pltpu.async_copy(rows_a, out_hbm.at[pl.ds(base + c0, CHUNK)], swa)
            hb.wait()
            pltpu.async_copy(rows_b, out_hbm.at[pl.ds(base + c1, CHUNK)], swb)

        pltpu.make_async_copy(rows_a, out_hbm.at[pl.ds(base, CHUNK)],
                              swa).wait()
        pltpu.make_async_copy(rows_b, out_hbm.at[pl.ds(base, CHUNK)],
                              swb).wait()

    return gather_k


def _make_scatter(nd_pad, ep):
    half = HID // 2
    e_sub = ep // SC_SUBCORES
    niter = e_sub // CHUNK
    rows_sub = nd_pad // SC_SUBCORES
    mesh = plsc.VectorSubcoreMesh(core_axis_name="c", subcore_axis_name="s")

    @functools.partial(
        pl.kernel,
        out_type=jax.ShapeDtypeStruct((2, nd_pad, half), jnp.float32),
        mesh=mesh,
        scratch_types=[pltpu.VMEM((CHUNK,), jnp.int32),
                       pltpu.VMEM((CHUNK,), jnp.int32),
                       pltpu.VMEM((CHUNK, half), jnp.float32),
                       pltpu.VMEM((CHUNK, half), jnp.float32),
                       pltpu.VMEM_SHARED((nd_pad, half), jnp.float32),
                       pltpu.SemaphoreType.DMA,
                       pltpu.SemaphoreType.DMA,
                       pltpu.SemaphoreType.DMA,
                       pltpu.SemaphoreType.DMA,
                       pltpu.SemaphoreType.DMA,
                       pltpu.SemaphoreType.DMA],
    )
    def scatter_k(m_hbm, di_hbm, zeros_hbm, out_hbm, idx_a, idx_b, rows_a,
                  rows_b, acc_sh, sia, sib, sla, slb, ssa, ssb):
        cid = lax.axis_index("c")
        sid = lax.axis_index("s")
        r0 = sid * rows_sub
        pltpu.sync_copy(zeros_hbm.at[pl.ds(r0, rows_sub)],
                        acc_sh.at[pl.ds(r0, rows_sub)])
        plsc.subcore_barrier()

        @pl.loop(0, niter // 2)
        def _(k):
            off = sid * e_sub + 2 * k * CHUNK
            hia = pltpu.async_copy(di_hbm.at[pl.ds(off, CHUNK)], idx_a, sia)
            hla = pltpu.async_copy(m_hbm.at[cid, pl.ds(off, CHUNK)],
                                   rows_a, sla)
            hib = pltpu.async_copy(di_hbm.at[pl.ds(off + CHUNK, CHUNK)],
                                   idx_b, sib)
            hlb = pltpu.async_copy(m_hbm.at[cid, pl.ds(off + CHUNK, CHUNK)],
                                   rows_b, slb)
            hia.wait()
            hla.wait()
            hsa = pltpu.async_copy(rows_a, acc_sh.at[idx_a], ssa, add=True)
            hib.wait()
            hlb.wait()
            hsb = pltpu.async_copy(rows_b, acc_sh.at[idx_b], ssb, add=True)
            hsa.wait()
            hsb.wait()

        plsc.subcore_barrier()
        pltpu.sync_copy(acc_sh.at[pl.ds(r0, rows_sub)],
                        out_hbm.at[cid, pl.ds(r0, rows_sub)])

    return scatter_k


_EPAD = {k: _ceil_to(v, PAD_UNIT) for k, v in EDGE_N.items()}
_GATHER = {k: _make_gather(ep) for k, ep in _EPAD.items()}
_SCATTER = {name: _make_scatter(NPAD[dt], _EPAD[name])
            for name, _, dt in CONVS}


def kernel(x_base, x_centroid, edge_attr_bb, edge_attr_bc, edge_attr_cc,
           edge_attr_cb, edge_weight_bb, edge_weight_bc, edge_weight_cc,
           edge_weight_cb, src_bb, dst_bb, src_bc, dst_bc, src_cc, dst_cc,
           src_cb, dst_cb, W_src, b_src, W_dst, b_dst, W_edge, b_edge, eps,
           W_mlp, b_mlp, ln_g, ln_b):
    ea = {"bb": edge_attr_bb, "bc": edge_attr_bc, "cc": edge_attr_cc,
          "cb": edge_attr_cb}
    ew = {"bb": edge_weight_bb, "bc": edge_weight_bc, "cc": edge_weight_cc,
          "cb": edge_weight_cb}
    si = {"bb": src_bb, "bc": src_bc, "cc": src_cc, "cb": src_cb}
    di = {"bb": dst_bb, "bc": dst_bc, "cc": dst_cc, "cb": dst_cb}

    for name in EDGE_N:
        e = EDGE_N[name]
        ep = _EPAD[name]
        pad = ep - e
        ea[name] = jnp.pad(ea[name], ((0, pad), (0, 0)))
        ew[name] = jnp.pad(ew[name], (0, pad)).reshape(ep, 1)
        si[name] = jnp.pad(si[name], (0, pad))
        di[name] = jnp.pad(di[name], (0, pad))

    zeros = {"base": jnp.zeros((NPAD["base"], HID // 2), jnp.float32),
             "centroid": jnp.zeros((NPAD["centroid"], HID // 2), jnp.float32)}

    xmap = {"base": x_base, "centroid": x_centroid}
    for l in range(L):
        outs = {"base": [], "centroid": []}
        for c, (name, st, dt) in enumerate(CONVS):
            xs = xmap[st]
            xd = xmap[dt]
            nd = NNODE[dt]
            blk_s = 1000 if xs.shape[0] == NB else NCN
            blk_d = 1000 if nd == NB else NCN
            hsrc = _lin_packed(xs, W_src[l, c], b_src[l, c], blk_s)
            hd = _lin(xd, W_dst[l, c] * (1.0 + eps[l, c]), b_dst[l, c], blk_d)
            g = _GATHER[name](hsrc, si[name])
            m = _msg(g, ea[name], ew[name], W_edge[l, c], b_edge[l, c], 2048)
            aggr = _SCATTER[name](m, di[name], zeros[dt])
            out = _post(aggr[0, :nd], aggr[1, :nd], hd, W_mlp[l, c, 0],
                        b_mlp[l, c, 0], ln_g[l, c], ln_b[l, c],
                        W_mlp[l, c, 1], b_mlp[l, c, 1], blk_d)
            outs[dt].append(out)
        xmap = {k: _combine(xmap[k], outs[k][0], outs[k][1],
                            1000 if NNODE[k] == NB else NCN)
                for k in xmap}
    return jnp.concatenate([xmap["base"], xmap["centroid"]], axis=0)


# speedup vs baseline: 2.3065x; 1.0357x over previous
import functools

import jax
import jax.numpy as jnp
from jax import lax
from jax.experimental import pallas as pl
from jax.experimental.pallas import tpu as pltpu
from jax.experimental.pallas import tpu_sc as plsc

HID = 256
NB = 10000
NCN = 1000
DE = 16
L = 2
CONVS = [("bb", "base", "base"), ("bc", "base", "centroid"),
         ("cc", "centroid", "centroid"), ("cb", "centroid", "base")]
EDGE_N = {"bb": 160000, "bc": 40000, "cc": 32000, "cb": 40000}

SC_CORES = 2
SC_SUBCORES = 16
CHUNK = 128
PAD_UNIT = SC_CORES * SC_SUBCORES * CHUNK

NPAD = {"base": 10240, "centroid": 1024}
NNODE = {"base": NB, "centroid": NCN}


def _ceil_to(x, m):
    return (x + m - 1) // m * m


def _lin_body(x_ref, w_ref, b_ref, o_ref):
    y = (jnp.dot(x_ref[...], w_ref[...],
                 preferred_element_type=jnp.float32) + b_ref[...])
    o_ref[...] = y.astype(o_ref.dtype)


def _lin(x, w, b, block_rows, out_dtype=jnp.float32):
    n, k = x.shape
    m = w.shape[1]
    return pl.pallas_call(
        _lin_body,
        grid=(n // block_rows,),
        in_specs=[pl.BlockSpec((block_rows, k), lambda i: (i, 0)),
                  pl.BlockSpec((k, m), lambda i: (0, 0)),
                  pl.BlockSpec((1, m), lambda i: (0, 0))],
        out_specs=pl.BlockSpec((block_rows, m), lambda i: (i, 0)),
        out_shape=jax.ShapeDtypeStruct((n, m), out_dtype),
    )(x, w, b.reshape(1, m))


def _pack_bf16_pair(lo, hi):
    def rne(x):
        u = jax.lax.bitcast_convert_type(x, jnp.uint32)
        return (u + 0x7FFF + ((u >> 16) & 1)) >> 16
    return jax.lax.bitcast_convert_type(rne(lo) | (rne(hi) << 16), jnp.int32)


def _unpack_bf16_pair(p):
    u = jax.lax.bitcast_convert_type(p, jnp.uint32)
    lo = jax.lax.bitcast_convert_type(u << 16, jnp.float32)
    hi = jax.lax.bitcast_convert_type(u & jnp.uint32(0xFFFF0000), jnp.float32)
    return lo, hi


def _lin_packed_body(x_ref, w_ref, b_ref, o_ref):
    y = (jnp.dot(x_ref[...], w_ref[...],
                 preferred_element_type=jnp.float32) + b_ref[...])
    half = y.shape[1] // 2
    o_ref[...] = _pack_bf16_pair(y[:, :half], y[:, half:])


def _lin_packed(x, w, b, block_rows):
    n, k = x.shape
    m = w.shape[1]
    return pl.pallas_call(
        _lin_packed_body,
        grid=(n // block_rows,),
        in_specs=[pl.BlockSpec((block_rows, k), lambda i: (i, 0)),
                  pl.BlockSpec((k, m), lambda i: (0, 0)),
                  pl.BlockSpec((1, m), lambda i: (0, 0))],
        out_specs=pl.BlockSpec((block_rows, m // 2), lambda i: (i, 0)),
        out_shape=jax.ShapeDtypeStruct((n, m // 2), jnp.int32),
    )(x, w, b.reshape(1, m))


def _msg_body(g_ref, ea_ref, ew_ref, w_ref, b_ref, o_ref):
    g_lo, g_hi = _unpack_bf16_pair(g_ref[...])
    pre = (jnp.concatenate([g_lo, g_hi], axis=1)
           + jnp.dot(ea_ref[...], w_ref[...],
                     preferred_element_type=jnp.float32) + b_ref[...])
    y = jax.nn.gelu(pre) * ew_ref[...]
    half = y.shape[1] // 2
    o_ref[0] = y[:, :half]
    o_ref[1] = y[:, half:]


def _msg(g, ea, ew, w, b, block_rows):
    ep = g.shape[0]
    return pl.pallas_call(
        _msg_body,
        grid=(ep // block_rows,),
        in_specs=[pl.BlockSpec((block_rows, HID // 2), lambda i: (i, 0)),
                  pl.BlockSpec((block_rows, DE), lambda i: (i, 0)),
                  pl.BlockSpec((block_rows, 1), lambda i: (i, 0)),
                  pl.BlockSpec((DE, HID), lambda i: (0, 0)),
                  pl.BlockSpec((1, HID), lambda i: (0, 0))],
        out_specs=pl.BlockSpec((2, block_rows, HID // 2), lambda i: (0, i, 0)),
        out_shape=jax.ShapeDtypeStruct((2, ep, HID // 2), jnp.float32),
    )(g, ea, ew, w, b.reshape(1, HID))


def _post_body(a0_ref, a1_ref, h_ref, m0_ref, b0_ref, g_ref, be_ref, m1_ref,
               b1_ref, o_ref):
    h = jnp.concatenate([a0_ref[...], a1_ref[...]], axis=1) + h_ref[...]
    h = jnp.dot(h, m0_ref[...], preferred_element_type=jnp.float32) + b0_ref[...]
    mu = jnp.mean(h, -1, keepdims=True)
    v = jnp.mean((h - mu) ** 2, -1, keepdims=True)
    h = (h - mu) / jnp.sqrt(v + 1e-5) * g_ref[...] + be_ref[...]
    h = jax.nn.gelu(h)
    o_ref[...] = (jnp.dot(h, m1_ref[...], preferred_element_type=jnp.float32)
                  + b1_ref[...])


def _post(a0, a1, hd, m0, b0, lg, lb, m1, b1, block_rows):
    n = a0.shape[0]
    full = lambda i: (0, 0)
    rows = lambda i: (i, 0)
    return pl.pallas_call(
        _post_body,
        grid=(n // block_rows,),
        in_specs=[pl.BlockSpec((block_rows, HID // 2), rows),
                  pl.BlockSpec((block_rows, HID // 2), rows),
                  pl.BlockSpec((block_rows, HID), rows),
                  pl.BlockSpec((HID, HID), full),
                  pl.BlockSpec((1, HID), full),
                  pl.BlockSpec((1, HID), full),
                  pl.BlockSpec((1, HID), full),
                  pl.BlockSpec((HID, HID), full),
                  pl.BlockSpec((1, HID), full)],
        out_specs=pl.BlockSpec((block_rows, HID), rows),
        out_shape=jax.ShapeDtypeStruct((n, HID), jnp.float32),
    )(a0, a1, hd, m0, b0.reshape(1, HID), lg.reshape(1, HID),
      lb.reshape(1, HID), m1, b1.reshape(1, HID))


def _combine_body(x_ref, a_ref, b_ref, o_ref):
    o_ref[...] = x_ref[...] + jax.nn.gelu(a_ref[...] + b_ref[...])


def _combine(x, a, b, block_rows):
    n = x.shape[0]
    rows = lambda i: (i, 0)
    return pl.pallas_call(
        _combine_body,
        grid=(n // block_rows,),
        in_specs=[pl.BlockSpec((block_rows, HID), rows)] * 3,
        out_specs=pl.BlockSpec((block_rows, HID), rows),
        out_shape=jax.ShapeDtypeStruct((n, HID), jnp.float32),
    )(x, a, b)


NBUF = 4


def _make_gather(ep):
    per_worker = ep // (SC_CORES * SC_SUBCORES)
    niter = per_worker // CHUNK
    main = niter // NBUF
    tail = niter % NBUF
    mesh = plsc.VectorSubcoreMesh(core_axis_name="c", subcore_axis_name="s")

    @functools.partial(
        pl.kernel,
        out_type=jax.ShapeDtypeStruct((ep, HID // 2), jnp.int32),
        mesh=mesh,
        scratch_types=([pltpu.VMEM((per_worker,), jnp.int32)]
                       + [pltpu.VMEM((CHUNK, HID // 2), jnp.int32)] * NBUF
                       + [pltpu.SemaphoreType.DMA] * (2 * NBUF)),
    )
    def gather_k(table_hbm, idx_hbm, out_hbm, idx_v, *bufs):
        rows = bufs[:NBUF]
        sg = bufs[NBUF:2 * NBUF]
        sw = bufs[2 * NBUF:]
        wid = lax.axis_index("s") * SC_CORES + lax.axis_index("c")
        base = wid * per_worker
        pltpu.sync_copy(idx_hbm.at[pl.ds(base, per_worker)], idx_v)

        def drain_wb(b):
            pltpu.make_async_copy(rows[b], out_hbm.at[pl.ds(base, CHUNK)],
                                  sw[b]).wait()

        @pl.loop(0, main)
        def _(k):
            c0 = NBUF * k * CHUNK
            hs = []
            for b in range(NBUF):
                @pl.when(k > 0)
                def _(b=b):
                    drain_wb(b)
                hs.append(pltpu.async_copy(
                    table_hbm.at[idx_v.at[pl.ds(c0 + b * CHUNK, CHUNK)]],
                    rows[b], sg[b]))
            for b in range(NBUF):
                hs[b].wait()
                pltpu.async_copy(rows[b],
                                 out_hbm.at[pl.ds(base + c0 + b * CHUNK,
                                                  CHUNK)], sw[b])

        hs = []
        for t in range(tail):
            if main > 0:
                drain_wb(t)
            c = (main * NBUF + t) * CHUNK
            hs.append(pltpu.async_copy(
                table_hbm.at[idx_v.at[pl.ds(c, CHUNK)]], rows[t], sg[t]))
        for t in range(tail):
            hs[t].wait()
            c = (main * NBUF + t) * CHUNK
            pltpu.async_copy(rows[t], out_hbm.at[pl.ds(base + c, CHUNK)],
                             sw[t])
        for b in range(NBUF if main > 0 else tail):
            drain_wb(b)

    return gather_k


def _make_scatter(nd_pad, ep):
    half = HID // 2
    schunk = 64
    e_sub = ep // SC_SUBCORES
    niter = e_sub // schunk
    rows_sub = nd_pad // SC_SUBCORES
    mesh = plsc.VectorSubcoreMesh(core_axis_name="c", subcore_axis_name="s")

    @functools.partial(
        pl.kernel,
        out_type=jax.ShapeDtypeStruct((2, nd_pad, half), jnp.float32),
        mesh=mesh,
        scratch_types=([pltpu.VMEM((schunk,), jnp.int32)] * NBUF
                       + [pltpu.VMEM((schunk, half), jnp.float32)] * NBUF
                       + [pltpu.VMEM_SHARED((nd_pad, half), jnp.float32)]
                       + [pltpu.SemaphoreType.DMA] * (2 * NBUF)),
    )
    def scatter_k(m_hbm, di_hbm, zeros_hbm, out_hbm, *bufs):
        idx = bufs[:NBUF]
        rows = bufs[NBUF:2 * NBUF]
        acc_sh = bufs[2 * NBUF]
        sl = bufs[2 * NBUF + 1:3 * NBUF + 1]
        ss = bufs[3 * NBUF + 1:]
        cid = lax.axis_index("c")
        sid = lax.axis_index("s")
        r0 = sid * rows_sub
        pltpu.sync_copy(zeros_hbm.at[pl.ds(r0, rows_sub)],
                        acc_sh.at[pl.ds(r0, rows_sub)])
        plsc.subcore_barrier()

        def drain_add(b):
            pltpu.make_async_copy(rows[b], acc_sh.at[idx[b]], ss[b]).wait()

        @pl.loop(0, niter // NBUF)
        def _(k):
            hs = []
            for b in range(NBUF):
                off = sid * e_sub + (NBUF * k + b) * schunk

                @pl.when(k > 0)
                def _(b=b):
                    drain_add(b)
                hi = pltpu.async_copy(di_hbm.at[pl.ds(off, schunk)], idx[b],
                                      sl[b])
                hr = pltpu.async_copy(m_hbm.at[cid, pl.ds(off, schunk)],
                                      rows[b], sl[b])
                hs.append((hi, hr))
            for b in range(NBUF):
                hi, hr = hs[b]
                hi.wait()
                hr.wait()
                pltpu.async_copy(rows[b], acc_sh.at[idx[b]], ss[b], add=True)

        for b in range(NBUF):
            drain_add(b)
        plsc.subcore_barrier()
        pltpu.sync_copy(acc_sh.at[pl.ds(r0, rows_sub)],
                        out_hbm.at[cid, pl.ds(r0, rows_sub)])

    return scatter_k


_EPAD = {k: _ceil_to(v, PAD_UNIT) for k, v in EDGE_N.items()}
_GATHER = {k: _make_gather(ep) for k, ep in _EPAD.items()}
_SCATTER = {name: _make_scatter(NPAD[dt], _EPAD[name])
            for name, _, dt in CONVS}


def kernel(x_base, x_centroid, edge_attr_bb, edge_attr_bc, edge_attr_cc,
           edge_attr_cb, edge_weight_bb, edge_weight_bc, edge_weight_cc,
           edge_weight_cb, src_bb, dst_bb, src_bc, dst_bc, src_cc, dst_cc,
           src_cb, dst_cb, W_src, b_src, W_dst, b_dst, W_edge, b_edge, eps,
           W_mlp, b_mlp, ln_g, ln_b):
    ea = {"bb": edge_attr_bb, "bc": edge_attr_bc, "cc": edge_attr_cc,
          "cb": edge_attr_cb}
    ew = {"bb": edge_weight_bb, "bc": edge_weight_bc, "cc": edge_weight_cc,
          "cb": edge_weight_cb}
    si = {"bb": src_bb, "bc": src_bc, "cc": src_cc, "cb": src_cb}
    di = {"bb": dst_bb, "bc": dst_bc, "cc": dst_cc, "cb": dst_cb}

    for name in EDGE_N:
        e = EDGE_N[name]
        ep = _EPAD[name]
        pad = ep - e
        ea[name] = jnp.pad(ea[name], ((0, pad), (0, 0)))
        ew[name] = jnp.pad(ew[name], (0, pad)).reshape(ep, 1)
        si[name] = jnp.pad(si[name], (0, pad))
        di[name] = jnp.pad(di[name], (0, pad))

    zeros = {"base": jnp.zeros((NPAD["base"], HID // 2), jnp.float32),
             "centroid": jnp.zeros((NPAD["centroid"], HID // 2), jnp.float32)}

    xmap = {"base": x_base, "centroid": x_centroid}
    for l in range(L):
        outs = {"base": [], "centroid": []}
        for c, (name, st, dt) in enumerate(CONVS):
            xs = xmap[st]
            xd = xmap[dt]
            nd = NNODE[dt]
            blk_s = 1000 if xs.shape[0] == NB else NCN
            blk_d = 1000 if nd == NB else NCN
            hsrc = _lin_packed(xs, W_src[l, c], b_src[l, c], blk_s)
            hd = _lin(xd, W_dst[l, c] * (1.0 + eps[l, c]), b_dst[l, c], blk_d)
            g = _GATHER[name](hsrc, si[name])
            m = _msg(g, ea[name], ew[name], W_edge[l, c], b_edge[l, c], 2048)
            aggr = _SCATTER[name](m, di[name], zeros[dt])
            out = _post(aggr[0, :nd], aggr[1, :nd], hd, W_mlp[l, c, 0],
                        b_mlp[l, c, 0], ln_g[l, c], ln_b[l, c],
                        W_mlp[l, c, 1], b_mlp[l, c, 1], blk_d)
            outs[dt].append(out)
        xmap = {k: _combine(xmap[k], outs[k][0], outs[k][1],
                            1000 if NNODE[k] == NB else NCN)
                for k in xmap}
    return jnp.concatenate([xmap["base"], xmap["centroid"]], axis=0)
